# Initial kernel scaffold; baseline (speedup 1.0000x reference)
import jax
import jax.numpy as jnp
from jax.experimental import pallas as pl


def kernel(x, edge_index, W1, b1, W2, b2):
    raise NotImplementedError("write your pallas kernel here")


# speedup vs baseline: 23.3442x; 23.3442x over previous
import functools

import jax
import jax.numpy as jnp
from jax import lax
from jax.experimental import pallas as pl
from jax.experimental.pallas import tpu as pltpu
from jax.experimental.pallas import tpu_sc as plsc

N_NODES = 10000
IN_DIM = 128
HID = 64
N_EDGES = 320000

NC = 2
NS = 16
NW = NC * NS
EPW = N_EDGES // NW
CHUNK = 80
NCHUNK = EPW // CHUNK
N_PAD = 10240
RPT = N_PAD // NS

_mesh = plsc.VectorSubcoreMesh(
    core_axis_name="c", subcore_axis_name="s", num_cores=NC, num_subcores=NS
)


@functools.partial(
    pl.kernel,
    out_type=jax.ShapeDtypeStruct((NC, N_NODES), jnp.float32),
    mesh=_mesh,
    scratch_types=[
        pltpu.VMEM((NCHUNK, CHUNK), jnp.int32),
        pltpu.VMEM((CHUNK,), jnp.float32),
        pltpu.VMEM_SHARED((N_NODES,), jnp.float32),
    ],
    compiler_params=pltpu.CompilerParams(use_tc_tiling_on_sc=False),
)
def _deg_kernel(dst_hbm, ones_hbm, zeros_hbm, out_hbm, idx_v, ones_v, acc):
    c = lax.axis_index("c")
    s = lax.axis_index("s")
    wid = s * NC + c
    pltpu.sync_copy(dst_hbm.at[wid], idx_v)
    pltpu.sync_copy(ones_hbm, ones_v)

    @pl.when(s == 0)
    def _():
        pltpu.sync_copy(zeros_hbm, acc)

    plsc.subcore_barrier()

    def body(j, carry):
        pltpu.sync_copy(ones_v, acc.at[idx_v.at[j]], add=True)
        return carry

    lax.fori_loop(0, NCHUNK, body, 0)
    plsc.subcore_barrier()

    @pl.when(s == 0)
    def _():
        pltpu.sync_copy(acc, out_hbm.at[c])


@functools.partial(
    pl.kernel,
    out_type=jax.ShapeDtypeStruct((NC, N_PAD, HID), jnp.float32),
    mesh=_mesh,
    scratch_types=[
        pltpu.VMEM((NCHUNK, CHUNK), jnp.int32),
        pltpu.VMEM((NCHUNK, CHUNK), jnp.int32),
        pltpu.VMEM((CHUNK, HID), jnp.float32),
        pltpu.VMEM_SHARED((N_PAD, HID), jnp.float32),
        pltpu.SemaphoreType.DMA,
    ],
    compiler_params=pltpu.CompilerParams(use_tc_tiling_on_sc=False),
)
def _agg_kernel(h_hbm, src_hbm, dst_hbm, zeros_hbm, out_hbm,
                src_v, dst_v, rows_v, acc, sem):
    c = lax.axis_index("c")
    s = lax.axis_index("s")
    wid = s * NC + c
    pltpu.sync_copy(src_hbm.at[wid], src_v)
    pltpu.sync_copy(dst_hbm.at[wid], dst_v)
    r0 = s * RPT
    pltpu.sync_copy(zeros_hbm.at[pl.ds(r0, RPT)], acc.at[pl.ds(r0, RPT)])
    plsc.subcore_barrier()

    def body(j, carry):
        pltpu.async_copy(h_hbm.at[src_v.at[j]], rows_v, sem).wait()
        pltpu.sync_copy(rows_v, acc.at[dst_v.at[j]], add=True)
        return carry

    lax.fori_loop(0, NCHUNK, body, 0)
    plsc.subcore_barrier()
    pltpu.sync_copy(acc.at[pl.ds(r0, RPT)], out_hbm.at[c, pl.ds(r0, RPT)])


def _tc1_body(x_ref, w_ref, d0_ref, d1_ref, dis_ref, hs_ref):
    deg = 1.0 + d0_ref[...] + d1_ref[...]
    dis = lax.rsqrt(deg)
    dis_ref[...] = dis
    hs_ref[...] = (
        jnp.dot(x_ref[...], w_ref[...], preferred_element_type=jnp.float32) * dis
    )


def _tc2_body(p0_ref, p1_ref, hs1_ref, dis_ref, b1_ref, w2_ref, hs2_ref):
    dis = dis_ref[...]
    h1 = jnp.maximum(
        dis * (p0_ref[...] + p1_ref[...] + hs1_ref[...]) + b1_ref[...], 0.0
    )
    hs2_ref[...] = (
        jnp.dot(h1, w2_ref[...], preferred_element_type=jnp.float32) * dis
    )


def _tc3_body(q0_ref, q1_ref, hs2_ref, dis_ref, b2_ref, out_ref):
    out_ref[...] = (
        dis_ref[...] * (q0_ref[...] + q1_ref[...] + hs2_ref[...]) + b2_ref[...]
    )


def kernel(x, edge_index, W1, b1, W2, b2):
    x = x.astype(jnp.float32)
    ei = edge_index.astype(jnp.int32)
    src3 = ei[0].reshape(NW, NCHUNK, CHUNK)
    dst3 = ei[1].reshape(NW, NCHUNK, CHUNK)
    ones = jnp.ones((CHUNK,), jnp.float32)
    zeros1 = jnp.zeros((N_NODES,), jnp.float32)
    zeros2 = jnp.zeros((N_PAD, HID), jnp.float32)

    degp = _deg_kernel(dst3, ones, zeros1)
    d0 = degp[0].reshape(N_NODES, 1)
    d1 = degp[1].reshape(N_NODES, 1)

    dis, hs1 = pl.pallas_call(
        _tc1_body,
        out_shape=[
            jax.ShapeDtypeStruct((N_NODES, 1), jnp.float32),
            jax.ShapeDtypeStruct((N_NODES, HID), jnp.float32),
        ],
    )(x, W1, d0, d1)

    p = _agg_kernel(hs1, src3, dst3, zeros2)[:, :N_NODES, :]

    hs2 = pl.pallas_call(
        _tc2_body,
        out_shape=jax.ShapeDtypeStruct((N_NODES, HID), jnp.float32),
    )(p[0], p[1], hs1, dis, b1.reshape(1, HID), W2)

    q = _agg_kernel(hs2, src3, dst3, zeros2)[:, :N_NODES, :]

    out = pl.pallas_call(
        _tc3_body,
        out_shape=jax.ShapeDtypeStruct((N_NODES, HID), jnp.float32),
    )(q[0], q[1], hs2, dis, b2.reshape(1, HID))
    return out


# speedup vs baseline: 33.2466x; 1.4242x over previous
import functools

import jax
import jax.numpy as jnp
from jax import lax
from jax.experimental import pallas as pl
from jax.experimental.pallas import tpu as pltpu
from jax.experimental.pallas import tpu_sc as plsc

N_NODES = 10000
IN_DIM = 128
HID = 64
N_EDGES = 320000

NC = 2
NS = 16
NW = NC * NS
EPW = N_EDGES // NW
CHUNK = 80
NCHUNK = EPW // CHUNK
N_PAD = 10240
RPT = N_PAD // NS

_mesh = plsc.VectorSubcoreMesh(
    core_axis_name="c", subcore_axis_name="s", num_cores=NC, num_subcores=NS
)


@functools.partial(
    pl.kernel,
    out_type=jax.ShapeDtypeStruct((NC, N_NODES), jnp.float32),
    mesh=_mesh,
    scratch_types=[
        pltpu.VMEM((NCHUNK, CHUNK), jnp.int32),
        pltpu.VMEM((CHUNK,), jnp.float32),
        pltpu.VMEM_SHARED((N_NODES,), jnp.float32),
    ],
    compiler_params=pltpu.CompilerParams(use_tc_tiling_on_sc=False),
)
def _deg_kernel(dst_hbm, ones_hbm, zeros_hbm, out_hbm, idx_v, ones_v, acc):
    c = lax.axis_index("c")
    s = lax.axis_index("s")
    wid = s * NC + c
    pltpu.sync_copy(dst_hbm.at[wid], idx_v)
    pltpu.sync_copy(ones_hbm, ones_v)

    @pl.when(s == 0)
    def _():
        pltpu.sync_copy(zeros_hbm, acc)

    plsc.subcore_barrier()

    def body(j, carry):
        pltpu.sync_copy(ones_v, acc.at[idx_v.at[j]], add=True)
        return carry

    lax.fori_loop(0, NCHUNK, body, 0)
    plsc.subcore_barrier()

    @pl.when(s == 0)
    def _():
        pltpu.sync_copy(acc, out_hbm.at[c])


@functools.partial(
    pl.kernel,
    out_type=jax.ShapeDtypeStruct((NC, N_PAD, HID), jnp.float32),
    mesh=_mesh,
    scratch_types=[
        pltpu.VMEM((NCHUNK, CHUNK), jnp.int32),
        pltpu.VMEM((NCHUNK, CHUNK), jnp.int32),
        pltpu.VMEM((CHUNK, HID), jnp.float32),
        pltpu.VMEM((CHUNK, HID), jnp.float32),
        pltpu.VMEM_SHARED((N_PAD, HID), jnp.float32),
        pltpu.SemaphoreType.DMA,
        pltpu.SemaphoreType.DMA,
    ],
    compiler_params=pltpu.CompilerParams(use_tc_tiling_on_sc=False),
)
def _agg_kernel(h_hbm, src_hbm, dst_hbm, zeros_hbm, out_hbm,
                src_v, dst_v, rows_a, rows_b, acc, gsem_a, gsem_b):
    c = lax.axis_index("c")
    s = lax.axis_index("s")
    wid = s * NC + c
    pltpu.sync_copy(src_hbm.at[wid], src_v)
    pltpu.sync_copy(dst_hbm.at[wid], dst_v)
    r0 = s * RPT
    pltpu.sync_copy(zeros_hbm.at[pl.ds(r0, RPT)], acc.at[pl.ds(r0, RPT)])
    plsc.subcore_barrier()

    def wait_gather(buf, sem, j):
        pltpu.make_async_copy(h_hbm.at[src_v.at[j]], buf, sem).wait()

    pltpu.async_copy(h_hbm.at[src_v.at[0]], rows_a, gsem_a)

    def body(k, carry):
        j0 = 2 * k
        pltpu.async_copy(h_hbm.at[src_v.at[j0 + 1]], rows_b, gsem_b)
        wait_gather(rows_a, gsem_a, j0)
        pltpu.sync_copy(rows_a, acc.at[dst_v.at[j0]], add=True)
        pltpu.async_copy(h_hbm.at[src_v.at[j0 + 2]], rows_a, gsem_a)
        wait_gather(rows_b, gsem_b, j0 + 1)
        pltpu.sync_copy(rows_b, acc.at[dst_v.at[j0 + 1]], add=True)
        return carry

    lax.fori_loop(0, (NCHUNK - 1) // 2, body, 0)
    wait_gather(rows_a, gsem_a, NCHUNK - 1)
    pltpu.sync_copy(rows_a, acc.at[dst_v.at[NCHUNK - 1]], add=True)
    plsc.subcore_barrier()
    pltpu.sync_copy(acc.at[pl.ds(r0, RPT)], out_hbm.at[c, pl.ds(r0, RPT)])


def _tc1_body(x_ref, w_ref, d0_ref, d1_ref, dis_ref, hs_ref):
    deg = 1.0 + d0_ref[...] + d1_ref[...]
    dis = lax.rsqrt(deg)
    dis_ref[...] = dis
    hs_ref[...] = (
        jnp.dot(x_ref[...], w_ref[...], preferred_element_type=jnp.float32) * dis
    )


def _tc2_body(p0_ref, p1_ref, hs1_ref, dis_ref, b1_ref, w2_ref, hs2_ref):
    dis = dis_ref[...]
    h1 = jnp.maximum(
        dis * (p0_ref[...] + p1_ref[...] + hs1_ref[...]) + b1_ref[...], 0.0
    )
    hs2_ref[...] = (
        jnp.dot(h1, w2_ref[...], preferred_element_type=jnp.float32) * dis
    )


def _tc3_body(q0_ref, q1_ref, hs2_ref, dis_ref, b2_ref, out_ref):
    out_ref[...] = (
        dis_ref[...] * (q0_ref[...] + q1_ref[...] + hs2_ref[...]) + b2_ref[...]
    )


def kernel(x, edge_index, W1, b1, W2, b2):
    x = x.astype(jnp.float32)
    ei = edge_index.astype(jnp.int32)
    src3 = ei[0].reshape(NW, NCHUNK, CHUNK)
    dst3 = ei[1].reshape(NW, NCHUNK, CHUNK)
    ones = jnp.ones((CHUNK,), jnp.float32)
    zeros1 = jnp.zeros((N_NODES,), jnp.float32)
    zeros2 = jnp.zeros((N_PAD, HID), jnp.float32)

    degp = _deg_kernel(dst3, ones, zeros1)
    d0 = degp[0].reshape(N_NODES, 1)
    d1 = degp[1].reshape(N_NODES, 1)

    dis, hs1 = pl.pallas_call(
        _tc1_body,
        out_shape=[
            jax.ShapeDtypeStruct((N_NODES, 1), jnp.float32),
            jax.ShapeDtypeStruct((N_NODES, HID), jnp.float32),
        ],
    )(x, W1, d0, d1)

    p = _agg_kernel(hs1, src3, dst3, zeros2)[:, :N_NODES, :]

    hs2 = pl.pallas_call(
        _tc2_body,
        out_shape=jax.ShapeDtypeStruct((N_NODES, HID), jnp.float32),
    )(p[0], p[1], hs1, dis, b1.reshape(1, HID), W2)

    q = _agg_kernel(hs2, src3, dst3, zeros2)[:, :N_NODES, :]

    out = pl.pallas_call(
        _tc3_body,
        out_shape=jax.ShapeDtypeStruct((N_NODES, HID), jnp.float32),
    )(q[0], q[1], hs2, dis, b2.reshape(1, HID))
    return out


# speedup vs baseline: 38.9640x; 1.1720x over previous
import functools

import jax
import jax.numpy as jnp
from jax import lax
from jax.experimental import pallas as pl
from jax.experimental.pallas import tpu as pltpu
from jax.experimental.pallas import tpu_sc as plsc

N_NODES = 10000
IN_DIM = 128
HID = 64
N_EDGES = 320000

NC = 2
NS = 16
NW = NC * NS
EPW = N_EDGES // NW
CHUNK = 80
NCHUNK = EPW // CHUNK
N_PAD = 10240
RPT = N_PAD // NS
NBUF = 4

_mesh = plsc.VectorSubcoreMesh(
    core_axis_name="c", subcore_axis_name="s", num_cores=NC, num_subcores=NS
)

@functools.partial(
    pl.kernel,
    out_type=jax.ShapeDtypeStruct((NC, N_NODES), jnp.float32),
    mesh=_mesh,
    scratch_types=[
        pltpu.VMEM((NCHUNK, CHUNK), jnp.int32),
        pltpu.VMEM((CHUNK,), jnp.float32),
        pltpu.VMEM((RPT,), jnp.float32),
        pltpu.VMEM_SHARED((N_PAD,), jnp.float32),
    ],
    compiler_params=pltpu.CompilerParams(use_tc_tiling_on_sc=False),
)
def _deg_kernel(dst_hbm, out_hbm, idx_v, ones_v, zrow_v, acc):
    c = lax.axis_index("c")
    s = lax.axis_index("s")
    wid = s * NC + c
    pltpu.sync_copy(dst_hbm.at[wid], idx_v)
    one16 = jnp.ones((16,), jnp.float32)
    zero16 = jnp.zeros((16,), jnp.float32)
    for i in range(CHUNK // 16):
        ones_v[pl.ds(i * 16, 16)] = one16

    def zbody(i, carry):
        zrow_v[pl.ds(i * 16, 16)] = zero16
        return carry

    lax.fori_loop(0, RPT // 16, zbody, 0)
    pltpu.sync_copy(zrow_v, acc.at[pl.ds(s * RPT, RPT)])
    plsc.subcore_barrier()

    def body(j, carry):
        pltpu.sync_copy(ones_v, acc.at[idx_v.at[j]], add=True)
        return carry

    lax.fori_loop(0, NCHUNK, body, 0)
    plsc.subcore_barrier()

    @pl.when(s == 0)
    def _():
        pltpu.sync_copy(acc.at[pl.ds(0, N_NODES)], out_hbm.at[c])


@functools.partial(
    pl.kernel,
    out_type=jax.ShapeDtypeStruct((NC, N_PAD, HID), jnp.float32),
    mesh=_mesh,
    scratch_types=[
        pltpu.VMEM((NCHUNK, CHUNK), jnp.int32),
        pltpu.VMEM((NCHUNK, CHUNK), jnp.int32),
        [pltpu.VMEM((CHUNK, HID), jnp.float32) for _ in range(NBUF)],
        pltpu.VMEM_SHARED((N_PAD, HID), jnp.float32),
        [pltpu.SemaphoreType.DMA for _ in range(NBUF)],
        [pltpu.SemaphoreType.DMA for _ in range(NBUF)],
    ],
    compiler_params=pltpu.CompilerParams(use_tc_tiling_on_sc=False),
)
def _agg_kernel(h_hbm, src_hbm, dst_hbm, out_hbm,
                src_v, dst_v, bufs, acc, gsems, ssems):
    c = lax.axis_index("c")
    s = lax.axis_index("s")
    wid = s * NC + c
    pltpu.sync_copy(src_hbm.at[wid], src_v)
    pltpu.sync_copy(dst_hbm.at[wid], dst_v)

    zero16 = jnp.zeros((16,), jnp.float32)

    def zbody(r, carry):
        for cc in range(HID // 16):
            bufs[0][r, pl.ds(cc * 16, 16)] = zero16
        return carry

    lax.fori_loop(0, CHUNK, zbody, 0)
    r0 = s * RPT
    for i in range(RPT // CHUNK):
        pltpu.sync_copy(bufs[0], acc.at[pl.ds(r0 + i * CHUNK, CHUNK)])
    plsc.subcore_barrier()

    def fire_gather(i, j):
        pltpu.async_copy(h_hbm.at[src_v.at[j]], bufs[i], gsems[i])

    def wait_gather(i, j):
        pltpu.make_async_copy(h_hbm.at[src_v.at[j]], bufs[i], gsems[i]).wait()

    for i in range(NBUF):
        fire_gather(i, i)

    def body(k, carry):
        j0 = NBUF * k
        descs = []
        for i in range(NBUF):
            wait_gather(i, j0 + i)
            descs.append(
                pltpu.async_copy(bufs[i], acc.at[dst_v.at[j0 + i]],
                                 ssems[i], add=True))
        for i in range(NBUF):
            descs[i].wait()
            jn = j0 + NBUF + i

            @pl.when(jn < NCHUNK)
            def _():
                fire_gather(i, jn)

        return carry

    lax.fori_loop(0, NCHUNK // NBUF, body, 0)
    tail = NBUF * (NCHUNK // NBUF)
    wait_gather(0, tail)
    pltpu.async_copy(bufs[0], acc.at[dst_v.at[tail]], ssems[0], add=True).wait()

    plsc.subcore_barrier()
    pltpu.sync_copy(acc.at[pl.ds(r0, RPT)], out_hbm.at[c, pl.ds(r0, RPT)])


def _tc1_body(x_ref, w_ref, d0_ref, d1_ref, dis_ref, hs_ref):
    deg = 1.0 + d0_ref[...] + d1_ref[...]
    dis = lax.rsqrt(deg)
    dis_ref[...] = dis
    hs_ref[...] = (
        jnp.dot(x_ref[...], w_ref[...], preferred_element_type=jnp.float32) * dis
    )


def _tc2_body(p0_ref, p1_ref, hs1_ref, dis_ref, b1_ref, w2_ref, hs2_ref):
    dis = dis_ref[...]
    h1 = jnp.maximum(
        dis * (p0_ref[...] + p1_ref[...] + hs1_ref[...]) + b1_ref[...], 0.0
    )
    hs2_ref[...] = (
        jnp.dot(h1, w2_ref[...], preferred_element_type=jnp.float32) * dis
    )


def _tc3_body(q0_ref, q1_ref, hs2_ref, dis_ref, b2_ref, out_ref):
    out_ref[...] = (
        dis_ref[...] * (q0_ref[...] + q1_ref[...] + hs2_ref[...]) + b2_ref[...]
    )


def kernel(x, edge_index, W1, b1, W2, b2):
    x = x.astype(jnp.float32)
    ei = edge_index.astype(jnp.int32)
    src3 = ei[0].reshape(NW, NCHUNK, CHUNK)
    dst3 = ei[1].reshape(NW, NCHUNK, CHUNK)

    degp = _deg_kernel(dst3)
    d0 = degp[0].reshape(N_NODES, 1)
    d1 = degp[1].reshape(N_NODES, 1)

    dis, hs1 = pl.pallas_call(
        _tc1_body,
        out_shape=[
            jax.ShapeDtypeStruct((N_NODES, 1), jnp.float32),
            jax.ShapeDtypeStruct((N_NODES, HID), jnp.float32),
        ],
    )(x, W1, d0, d1)

    p = _agg_kernel(hs1, src3, dst3)[:, :N_NODES, :]

    hs2 = pl.pallas_call(
        _tc2_body,
        out_shape=jax.ShapeDtypeStruct((N_NODES, HID), jnp.float32),
    )(p[0], p[1], hs1, dis, b1.reshape(1, HID), W2)

    q = _agg_kernel(hs2, src3, dst3)[:, :N_NODES, :]

    out = pl.pallas_call(
        _tc3_body,
        out_shape=jax.ShapeDtypeStruct((N_NODES, HID), jnp.float32),
    )(q[0], q[1], hs2, dis, b2.reshape(1, HID))
    return out
